# baseline (speedup 1.0000x reference)
import functools

import jax
import jax.numpy as jnp
from jax import lax
from jax.experimental import pallas as pl
from jax.experimental.pallas import tpu as pltpu
from jax.experimental.pallas import tpu_sc as plsc

N = 10000
E = 320000
ED = 16
H = 128
OUT = 128
NUM_LAYERS = 3
EPS = 1e-5

NC = 2
NS = 16
NW = NC * NS
ROWS = E // 128
RPW = ROWS // NW
TAIL = ROWS - RPW * NW
STR = 624

_f32 = jnp.float32


def _mesh():
    return plsc.VectorSubcoreMesh(core_axis_name="c", subcore_axis_name="s")


def _striped_copy(s, src, dst):
    pltpu.sync_copy(src.at[pl.ds(s * STR, STR)], dst.at[pl.ds(s * STR, STR)])

    @pl.when(s == NS - 1)
    def _():
        rem = N - NS * STR
        pltpu.sync_copy(src.at[pl.ds(NS * STR, rem)], dst.at[pl.ds(NS * STR, rem)])


def _stats_body(ea_ref, sum_ref, sq_ref):
    i = pl.program_id(0)
    x = ea_ref[...]

    @pl.when(i == 0)
    def _():
        sum_ref[...] = jnp.zeros_like(sum_ref)
        sq_ref[...] = jnp.zeros_like(sq_ref)

    sum_ref[...] += jnp.sum(x, axis=1, keepdims=True)
    sq_ref[...] += jnp.sum(x * x, axis=1, keepdims=True)


def _stats(eaT):
    bs = 16000
    return pl.pallas_call(
        _stats_body,
        grid=(E // bs,),
        in_specs=[pl.BlockSpec((ED, bs), lambda i: (0, i))],
        out_specs=[pl.BlockSpec((ED, 1), lambda i: (0, 0))] * 2,
        out_shape=[jax.ShapeDtypeStruct((ED, 1), _f32)] * 2,
    )(eaT)


KG = 3
HROWS = ROWS // 2
GW = HROWS // NW
GTAIL = HROWS - GW * NW
GIPW = GW * 128


def _gather_body(p_hbm, src_hbm, g_hbm, gb0, gb1, ibufall, gsem, wsem, *, half):
    c = lax.axis_index("c")
    s = lax.axis_index("s")
    w = c * NS + s
    base = w * GW
    hoff = half * HROWS
    gbufs = (gb0, gb1)

    pltpu.sync_copy(src_hbm.at[pl.ds((hoff + base) * 128, GIPW)],
                    ibufall.at[pl.ds(0, GIPW)])

    @pl.when(w < GTAIL)
    def _():
        pltpu.sync_copy(src_hbm.at[pl.ds((hoff + NW * GW + w) * 128, 128)],
                        ibufall.at[pl.ds(GIPW, 128)])

    def g_descs(k, b):
        return [
            pltpu.make_async_copy(
                p_hbm.at[ibufall.at[pl.ds((k * KG + j) * 128, 128)]],
                gbufs[b].at[pl.ds(j * 128, 128)], gsem)
            for j in range(KG)
        ]

    def wb_desc(k, b):
        return pltpu.make_async_copy(
            gbufs[b], g_hbm.at[pl.ds((base + k * KG) * 128, KG * 128)], wsem)

    nch = GW // KG
    for k in range(nch):
        b = k % 2
        if k >= 2:
            wb_desc(k - 2, b).wait()
        descs = g_descs(k, b)
        for d in descs:
            d.start()
        for d in descs:
            d.wait()
        wb_desc(k, b).start()
    for k in (nch - 2, nch - 1):
        wb_desc(k, k % 2).wait()

    @pl.when(w < GTAIL)
    def _tail():
        r = NW * GW + w
        d = pltpu.make_async_copy(
            p_hbm.at[ibufall.at[pl.ds(GIPW, 128)]],
            gbufs[0].at[pl.ds(0, 128)], gsem)
        d.start()
        d.wait()
        pltpu.sync_copy(gbufs[0].at[pl.ds(0, 128)], g_hbm.at[pl.ds(r * 128, 128)])


def _gather(p, src1d, half):
    import functools as _ft
    fn = pl.kernel(
        _ft.partial(_gather_body, half=half),
        out_type=jax.ShapeDtypeStruct((HROWS * 128, H), _f32),
        mesh=_mesh(),
        scratch_types=[
            pltpu.VMEM((KG * 128, H), _f32),
            pltpu.VMEM((KG * 128, H), _f32),
            pltpu.VMEM((GIPW + 128,), jnp.int32),
            pltpu.SemaphoreType.DMA,
            pltpu.SemaphoreType.DMA,
        ],
    )
    return fn(p, src1d)


KC = 1
SW = HROWS // NS
STAILC = HROWS - SW * NS


def _scatter_body(msg0_hbm, msg1_hbm, dst3_hbm, zero128_hbm, agg_hbm,
                  mb0, mb1, ib0, ib1, acc, lsem, ssem):
    c = lax.axis_index("c")
    s = lax.axis_index("s")
    mbufs = (mb0, mb1)
    ibufs = (ib0, ib1)

    _striped_copy(s, zero128_hbm, acc)
    plsc.subcore_barrier()

    def emit(msg_hbm, row_base):
        base = s * SW

        def load_descs(k, b):
            return [
                pltpu.make_async_copy(
                    dst3_hbm.at[pl.ds(row_base + base + k, 1)], ibufs[b], lsem),
                pltpu.make_async_copy(
                    msg_hbm.at[pl.ds((base + k) * 128, 128)], mbufs[b], lsem),
            ]

        def add_desc(k, b):
            return pltpu.make_async_copy(mbufs[b], acc.at[ibufs[b].at[0, 0]], ssem)

        for b in range(2):
            for d in load_descs(b, b):
                d.start()

        @pl.loop(0, SW // 2)
        def _o(o):
            for b in range(2):
                k = o * 2 + b
                for d in load_descs(k, b):
                    d.wait()
                d = add_desc(k, b)
                d.start(add=True)
                d.wait()

                @pl.when(k + 2 < SW)
                def _():
                    for d2 in load_descs(k + 2, b):
                        d2.start()

        @pl.when(s < STAILC)
        def _tail():
            r = NS * SW + s
            pltpu.sync_copy(dst3_hbm.at[pl.ds(row_base + r, 1)], ib0)
            pltpu.sync_copy(msg_hbm.at[pl.ds(r * 128, 128)], mb0)
            pltpu.sync_copy(mb0, acc.at[ib0.at[0, 0]], add=True)

    @pl.when(c == 0)
    def _h0():
        emit(msg0_hbm, 0)

    @pl.when(c == 1)
    def _h1():
        emit(msg1_hbm, HROWS)

    plsc.subcore_barrier()
    _striped_copy(s, acc, agg_hbm.at[c])


def _scatter(msg0, msg1, dst3, zeros128):
    fn = pl.kernel(
        _scatter_body,
        out_type=jax.ShapeDtypeStruct((NC, N, H), _f32),
        mesh=_mesh(),
        scratch_types=[
            pltpu.VMEM((128, H), _f32),
            pltpu.VMEM((128, H), _f32),
            pltpu.VMEM((1, 1, 128), jnp.int32),
            pltpu.VMEM((1, 1, 128), jnp.int32),
            pltpu.VMEM_SHARED((N, H), _f32),
            pltpu.SemaphoreType.DMA,
            pltpu.SemaphoreType.DMA,
        ],
    )
    return fn(msg0, msg1, dst3, zeros128)


CR = 8
CW = 1280
HN = CR * CW // 2
IPW = RPW * 128


def _counts_body(dst_hbm, cnt_hbm, ibufall, cbuf, obuf):
    c = lax.axis_index("c")
    s = lax.axis_index("s")
    w = c * NS + s
    lane = lax.iota(jnp.int32, 16)
    ones = jnp.ones((16,), jnp.int32)
    zeros = jnp.zeros((16,), jnp.int32)

    pltpu.sync_copy(dst_hbm.at[pl.ds(w * IPW, IPW)], ibufall.at[pl.ds(0, IPW)])

    @pl.when(w < TAIL)
    def _():
        pltpu.sync_copy(dst_hbm.at[pl.ds(NW * IPW + w * 128, 128)],
                        ibufall.at[pl.ds(IPW, 128)])

    for half in range(2):
        lo = half * HN

        @pl.loop(0, HN, unroll=8)
        def _zero(i):
            cbuf[pl.ds(i * 16, 16)] = zeros

        def _count(i):
            idx = ibufall[pl.ds(i * 16, 16)]
            rel = idx - lo
            m = (rel >= 0) & (rel < HN)
            addr = rel * 16 + lane
            plsc.addupdate_scatter(cbuf, [addr], ones, mask=m)

        pl.loop(0, IPW // 16, unroll=4)(_count)

        @pl.when(w < TAIL)
        def _count_tail():
            pl.loop(IPW // 16, (IPW + 128) // 16)(_count)

        @pl.loop(0, HN // 16, unroll=2)
        def _reduce(gi):
            n0 = gi * 16
            base = n0 * 16 + lane * 16
            acc = plsc.load_gather(cbuf, [base])
            for l in range(1, 16):
                acc = acc + plsc.load_gather(cbuf, [base + l])
            flat = lo + n0
            obuf[flat // CW, pl.ds(flat % CW, 16)] = acc

    pltpu.sync_copy(obuf, cnt_hbm.at[w])


def _counts(dst1d):
    fn = pl.kernel(
        _counts_body,
        out_type=jax.ShapeDtypeStruct((NW, CR, CW), jnp.int32),
        mesh=_mesh(),
        scratch_types=[
            pltpu.VMEM((IPW + 128,), jnp.int32),
            pltpu.VMEM((HN * 16,), jnp.int32),
            pltpu.VMEM((CR, CW), jnp.int32),
        ],
        compiler_params=pltpu.CompilerParams(needs_layout_passes=False),
    )
    return fn(dst1d)


def _dotT(a, b):
    return lax.dot_general(a, b, ((((0,), (0,))), ((), ())),
                           preferred_element_type=_f32)


def _qinit_body(ea_ref, Wi_ref, q_ref):
    q_ref[...] = _dotT(ea_ref[...], Wi_ref[...])


def _qinit(eaT, Wi, half):
    nb = E // BM // 2
    return pl.pallas_call(
        _qinit_body,
        grid=(nb,),
        in_specs=[
            pl.BlockSpec((ED, BM), lambda i: (0, i + half * nb)),
            pl.BlockSpec((ED, H), lambda i: (0, 0)),
        ],
        out_specs=pl.BlockSpec((BM, H), lambda i: (i, 0)),
        out_shape=jax.ShapeDtypeStruct((E // 2, H), _f32),
    )(eaT, Wi)


def _prep_body(qagg_ref, cnt_ref, bit_ref, bi_ref, W1b_ref, p_ref, cinv_ref):
    cnt = cnt_ref[...]
    rin = 1.0 / jnp.maximum(cnt, 1.0)
    nz = (cnt > 0.0).astype(_f32)
    x0 = jax.nn.relu(
        (qagg_ref[0] + qagg_ref[1]) * rin + nz * bit_ref[...] + bi_ref[...])
    p_ref[...] = jnp.dot(x0, W1b_ref[...], preferred_element_type=_f32)
    cinv_ref[...] = rin


def _prep(qagg, cnt, bit, bi, W1b):
    return pl.pallas_call(
        _prep_body,
        out_shape=[
            jax.ShapeDtypeStruct((N, H), _f32),
            jax.ShapeDtypeStruct((N, 1), _f32),
        ],
    )(qagg, cnt, bit, bi, W1b)


BM = 6400


def _msg_body(ea_ref, g_ref, W1_ref, b1_ref, W2_ref, b2_ref, out_ref):
    h = jax.nn.relu(_dotT(ea_ref[...], W1_ref[...]) + g_ref[...] + b1_ref[...])
    out_ref[...] = jax.nn.relu(
        jnp.dot(h, W2_ref[...], preferred_element_type=_f32) + b2_ref[...])


def _msg(eaT, g, W1ap, b1p, We2, be2, half):
    nb = E // BM // 2
    return pl.pallas_call(
        _msg_body,
        grid=(nb,),
        in_specs=[
            pl.BlockSpec((ED, BM), lambda i: (0, i + half * nb)),
            pl.BlockSpec((BM, H), lambda i: (i, 0)),
            pl.BlockSpec((ED, H), lambda i: (0, 0)),
            pl.BlockSpec((1, H), lambda i: (0, 0)),
            pl.BlockSpec((H, H), lambda i: (0, 0)),
            pl.BlockSpec((1, H), lambda i: (0, 0)),
        ],
        out_specs=pl.BlockSpec((BM, H), lambda i: (i, 0)),
        out_shape=jax.ShapeDtypeStruct((E // 2, H), _f32),
    )(eaT, g, W1ap, b1p, We2, be2)


def _update_body(aggp_ref, cinv_ref, Wu_ref, bu_ref, Wn_ref, bn_ref, out_ref):
    agg = (aggp_ref[0] + aggp_ref[1]) * cinv_ref[...]
    x = jax.nn.relu(
        jnp.dot(agg, Wu_ref[...], preferred_element_type=_f32) + bu_ref[...])
    out_ref[...] = jnp.dot(x, Wn_ref[...], preferred_element_type=_f32) + bn_ref[...]


def _update(aggp, cinv, Wu, bu, Wn, bn):
    return pl.pallas_call(
        _update_body,
        out_shape=jax.ShapeDtypeStruct((N, Wn.shape[1]), _f32),
    )(aggp, cinv, Wu, bu, Wn, bn)


def kernel(edge_index, edge_attr, bn_gamma, bn_beta, W_init, b_init,
           W_e1, b_e1, W_e2, b_e2, W_u, b_u, W_out, b_out):
    src1d = edge_index[0]
    dst3 = edge_index[1].reshape(ROWS, 1, 128)

    eaT = edge_attr.T
    ssum, ssq = _stats(eaT)
    mu = ssum.reshape(1, ED) / float(E)
    var = ssq.reshape(1, ED) / float(E) - mu * mu
    sv = bn_gamma[None, :] * lax.rsqrt(var + EPS)
    tv = bn_beta[None, :] - mu * sv
    W1a = W_e1[:ED]
    W1b = W_e1[ED:]
    W1ap = W1a * sv.reshape(ED, 1)
    b1p = tv @ W1a + b_e1[None, :]
    Wi = W_init * sv.reshape(ED, 1)
    bit = tv @ W_init

    zeros128 = jnp.zeros((N, H), _f32)
    q0 = _qinit(eaT, Wi, 0)
    q1 = _qinit(eaT, Wi, 1)
    qagg = _scatter(q0, q1, dst3, zeros128)
    cnts = _counts(edge_index[1])
    cnt = cnts.sum(axis=0).reshape(-1)[:N].astype(_f32).reshape(N, 1)

    p, cinv = _prep(qagg, cnt, bit, b_init[None, :], W1b)

    be2 = b_e2[None, :]
    bu = b_u[None, :]
    for layer in range(NUM_LAYERS):
        g0 = _gather(p, src1d, 0)
        g1 = _gather(p, src1d, 1)
        m0 = _msg(eaT, g0, W1ap, b1p, W_e2, be2, 0)
        m1 = _msg(eaT, g1, W1ap, b1p, W_e2, be2, 1)
        aggp = _scatter(m0, m1, dst3, zeros128)
        if layer < NUM_LAYERS - 1:
            p = _update(aggp, cinv, W_u, bu, W1b, jnp.zeros((1, H), _f32))
        else:
            out = _update(aggp, cinv, W_u, bu, W_out, b_out[None, :])
    return out


# speedup vs baseline: 1.0305x; 1.0305x over previous
import functools

import jax
import jax.numpy as jnp
from jax import lax
from jax.experimental import pallas as pl
from jax.experimental.pallas import tpu as pltpu
from jax.experimental.pallas import tpu_sc as plsc

N = 10000
E = 320000
ED = 16
H = 128
OUT = 128
NUM_LAYERS = 3
EPS = 1e-5

NC = 2
NS = 16
NW = NC * NS
ROWS = E // 128
RPW = ROWS // NW
TAIL = ROWS - RPW * NW
STR = 624

_f32 = jnp.float32


def _mesh():
    return plsc.VectorSubcoreMesh(core_axis_name="c", subcore_axis_name="s")


def _striped_copy(s, src, dst):
    pltpu.sync_copy(src.at[pl.ds(s * STR, STR)], dst.at[pl.ds(s * STR, STR)])

    @pl.when(s == NS - 1)
    def _():
        rem = N - NS * STR
        pltpu.sync_copy(src.at[pl.ds(NS * STR, rem)], dst.at[pl.ds(NS * STR, rem)])


def _stats_body(ea_ref, sum_ref, sq_ref):
    i = pl.program_id(0)
    x = ea_ref[...]

    @pl.when(i == 0)
    def _():
        sum_ref[...] = jnp.zeros_like(sum_ref)
        sq_ref[...] = jnp.zeros_like(sq_ref)

    sum_ref[...] += jnp.sum(x, axis=1, keepdims=True)
    sq_ref[...] += jnp.sum(x * x, axis=1, keepdims=True)


def _stats(eaT):
    bs = 16000
    return pl.pallas_call(
        _stats_body,
        grid=(E // bs,),
        in_specs=[pl.BlockSpec((ED, bs), lambda i: (0, i))],
        out_specs=[pl.BlockSpec((ED, 1), lambda i: (0, 0))] * 2,
        out_shape=[jax.ShapeDtypeStruct((ED, 1), _f32)] * 2,
    )(eaT)


KG = 3
HROWS = ROWS // 2
GW = HROWS // NW
GTAIL = HROWS - GW * NW
GIPW = GW * 128


def _gather_body(p_hbm, src_hbm, g_hbm, gb0, gb1, ibufall, gsem, wsem, *, half):
    c = lax.axis_index("c")
    s = lax.axis_index("s")
    w = c * NS + s
    base = w * GW
    hoff = half * HROWS
    gbufs = (gb0, gb1)

    pltpu.sync_copy(src_hbm.at[pl.ds((hoff + base) * 128, GIPW)],
                    ibufall.at[pl.ds(0, GIPW)])

    @pl.when(w < GTAIL)
    def _():
        pltpu.sync_copy(src_hbm.at[pl.ds((hoff + NW * GW + w) * 128, 128)],
                        ibufall.at[pl.ds(GIPW, 128)])

    def g_descs(k, b):
        return [
            pltpu.make_async_copy(
                p_hbm.at[ibufall.at[pl.ds((k * KG + j) * 128, 128)]],
                gbufs[b].at[pl.ds(j * 128, 128)], gsem)
            for j in range(KG)
        ]

    def wb_desc(k, b):
        return pltpu.make_async_copy(
            gbufs[b], g_hbm.at[pl.ds((base + k * KG) * 128, KG * 128)], wsem)

    nch = GW // KG
    for k in range(nch):
        b = k % 2
        if k >= 2:
            wb_desc(k - 2, b).wait()
        descs = g_descs(k, b)
        for d in descs:
            d.start()
        for d in descs:
            d.wait()
        wb_desc(k, b).start()
    for k in (nch - 2, nch - 1):
        wb_desc(k, k % 2).wait()

    @pl.when(w < GTAIL)
    def _tail():
        r = NW * GW + w
        d = pltpu.make_async_copy(
            p_hbm.at[ibufall.at[pl.ds(GIPW, 128)]],
            gbufs[0].at[pl.ds(0, 128)], gsem)
        d.start()
        d.wait()
        pltpu.sync_copy(gbufs[0].at[pl.ds(0, 128)], g_hbm.at[pl.ds(r * 128, 128)])


def _gather(p, src1d, half):
    import functools as _ft
    fn = pl.kernel(
        _ft.partial(_gather_body, half=half),
        out_type=jax.ShapeDtypeStruct((HROWS * 128, H), _f32),
        mesh=_mesh(),
        scratch_types=[
            pltpu.VMEM((KG * 128, H), _f32),
            pltpu.VMEM((KG * 128, H), _f32),
            pltpu.VMEM((GIPW + 128,), jnp.int32),
            pltpu.SemaphoreType.DMA,
            pltpu.SemaphoreType.DMA,
        ],
    )
    return fn(p, src1d)


KC = 1
SW = HROWS // NS
STAILC = HROWS - SW * NS


def _scatter_body(msg0_hbm, msg1_hbm, dst3_hbm, zero128_hbm, agg_hbm,
                  mb0, mb1, ib0, ib1, acc, lsem, ssem):
    c = lax.axis_index("c")
    s = lax.axis_index("s")
    mbufs = (mb0, mb1)
    ibufs = (ib0, ib1)

    _striped_copy(s, zero128_hbm, acc)
    plsc.subcore_barrier()

    def emit(msg_hbm, row_base):
        base = s * SW

        def load_descs(k, b):
            return [
                pltpu.make_async_copy(
                    dst3_hbm.at[pl.ds(row_base + base + k, 1)], ibufs[b], lsem),
                pltpu.make_async_copy(
                    msg_hbm.at[pl.ds((base + k) * 128, 128)], mbufs[b], lsem),
            ]

        def add_desc(k, b):
            return pltpu.make_async_copy(mbufs[b], acc.at[ibufs[b].at[0, 0]], ssem)

        for b in range(2):
            for d in load_descs(b, b):
                d.start()

        @pl.loop(0, SW // 2)
        def _o(o):
            for b in range(2):
                k = o * 2 + b
                for d in load_descs(k, b):
                    d.wait()
                d = add_desc(k, b)
                d.start(add=True)
                d.wait()

                @pl.when(k + 2 < SW)
                def _():
                    for d2 in load_descs(k + 2, b):
                        d2.start()

        @pl.when(s < STAILC)
        def _tail():
            r = NS * SW + s
            pltpu.sync_copy(dst3_hbm.at[pl.ds(row_base + r, 1)], ib0)
            pltpu.sync_copy(msg_hbm.at[pl.ds(r * 128, 128)], mb0)
            pltpu.sync_copy(mb0, acc.at[ib0.at[0, 0]], add=True)

    @pl.when(c == 0)
    def _h0():
        emit(msg0_hbm, 0)

    @pl.when(c == 1)
    def _h1():
        emit(msg1_hbm, HROWS)

    plsc.subcore_barrier()
    _striped_copy(s, acc, agg_hbm.at[c])


def _scatter(msg0, msg1, dst3, zeros128):
    fn = pl.kernel(
        _scatter_body,
        out_type=jax.ShapeDtypeStruct((NC, N, H), _f32),
        mesh=_mesh(),
        scratch_types=[
            pltpu.VMEM((128, H), _f32),
            pltpu.VMEM((128, H), _f32),
            pltpu.VMEM((1, 1, 128), jnp.int32),
            pltpu.VMEM((1, 1, 128), jnp.int32),
            pltpu.VMEM_SHARED((N, H), _f32),
            pltpu.SemaphoreType.DMA,
            pltpu.SemaphoreType.DMA,
        ],
    )
    return fn(msg0, msg1, dst3, zeros128)


KS = 13


def _seg16_body(attr_hbm, dst3_hbm, zero16_hbm, outS_hbm, abuf, ibuf, accS):
    c = lax.axis_index("c")
    s = lax.axis_index("s")
    w = c * NS + s
    _striped_copy(s, zero16_hbm, accS)
    plsc.subcore_barrier()
    base = w * RPW

    @pl.loop(0, RPW // KS)
    def _chunk(k):
        r0 = base + k * KS
        pltpu.sync_copy(dst3_hbm.at[pl.ds(r0, KS)], ibuf)
        pltpu.sync_copy(attr_hbm.at[pl.ds(r0 * 128, KS * 128)], abuf)
        for j in range(KS):
            pltpu.sync_copy(abuf.at[pl.ds(j * 128, 128)], accS.at[ibuf.at[j, 0]], add=True)

    @pl.when(w < TAIL)
    def _tail():
        r = NW * RPW + w
        pltpu.sync_copy(dst3_hbm.at[pl.ds(r, 1)], ibuf.at[pl.ds(0, 1)])
        pltpu.sync_copy(attr_hbm.at[pl.ds(r * 128, 128)], abuf.at[pl.ds(0, 128)])
        pltpu.sync_copy(abuf.at[pl.ds(0, 128)], accS.at[ibuf.at[0, 0]], add=True)

    plsc.subcore_barrier()
    _striped_copy(s, accS, outS_hbm.at[c])


def _seg16(edge_attr, dst3, zeros16):
    fn = pl.kernel(
        _seg16_body,
        out_type=jax.ShapeDtypeStruct((NC, N, ED), _f32),
        mesh=_mesh(),
        scratch_types=[
            pltpu.VMEM((KS * 128, ED), _f32),
            pltpu.VMEM((KS, 1, 128), jnp.int32),
            pltpu.VMEM_SHARED((N, ED), _f32),
        ],
        compiler_params=pltpu.CompilerParams(use_tc_tiling_on_sc=False),
    )
    return fn(edge_attr, dst3, zeros16)


CR = 8
CW = 1280
HN = CR * CW // 2
IPW = RPW * 128


def _counts_body(dst_hbm, cnt_hbm, ibufall, cbuf, obuf):
    c = lax.axis_index("c")
    s = lax.axis_index("s")
    w = c * NS + s
    lane = lax.iota(jnp.int32, 16)
    ones = jnp.ones((16,), jnp.int32)
    zeros = jnp.zeros((16,), jnp.int32)

    pltpu.sync_copy(dst_hbm.at[pl.ds(w * IPW, IPW)], ibufall.at[pl.ds(0, IPW)])

    @pl.when(w < TAIL)
    def _():
        pltpu.sync_copy(dst_hbm.at[pl.ds(NW * IPW + w * 128, 128)],
                        ibufall.at[pl.ds(IPW, 128)])

    for half in range(2):
        lo = half * HN

        @pl.loop(0, HN, unroll=8)
        def _zero(i):
            cbuf[pl.ds(i * 16, 16)] = zeros

        def _count(i):
            idx = ibufall[pl.ds(i * 16, 16)]
            rel = idx - lo
            m = (rel >= 0) & (rel < HN)
            addr = rel * 16 + lane
            plsc.addupdate_scatter(cbuf, [addr], ones, mask=m)

        pl.loop(0, IPW // 16, unroll=4)(_count)

        @pl.when(w < TAIL)
        def _count_tail():
            pl.loop(IPW // 16, (IPW + 128) // 16)(_count)

        @pl.loop(0, HN // 16, unroll=2)
        def _reduce(gi):
            n0 = gi * 16
            base = n0 * 16 + lane * 16
            acc = plsc.load_gather(cbuf, [base])
            for l in range(1, 16):
                acc = acc + plsc.load_gather(cbuf, [base + l])
            flat = lo + n0
            obuf[flat // CW, pl.ds(flat % CW, 16)] = acc

    pltpu.sync_copy(obuf, cnt_hbm.at[w])


def _counts(dst1d):
    fn = pl.kernel(
        _counts_body,
        out_type=jax.ShapeDtypeStruct((NW, CR, CW), jnp.int32),
        mesh=_mesh(),
        scratch_types=[
            pltpu.VMEM((IPW + 128,), jnp.int32),
            pltpu.VMEM((HN * 16,), jnp.int32),
            pltpu.VMEM((CR, CW), jnp.int32),
        ],
        compiler_params=pltpu.CompilerParams(needs_layout_passes=False),
    )
    return fn(dst1d)


def _dotT(a, b):
    return lax.dot_general(a, b, ((((0,), (0,))), ((), ())),
                           preferred_element_type=_f32)


def _qinit_body(ea_ref, Wi_ref, q_ref):
    q_ref[...] = _dotT(ea_ref[...], Wi_ref[...])


def _qinit(eaT, Wi, half):
    nb = E // BM // 2
    return pl.pallas_call(
        _qinit_body,
        grid=(nb,),
        in_specs=[
            pl.BlockSpec((ED, BM), lambda i: (0, i + half * nb)),
            pl.BlockSpec((ED, H), lambda i: (0, 0)),
        ],
        out_specs=pl.BlockSpec((BM, H), lambda i: (i, 0)),
        out_shape=jax.ShapeDtypeStruct((E // 2, H), _f32),
    )(eaT, Wi)


def _prep_body(segp_ref, cnt_ref, Wi_ref, bit_ref, bi_ref, W1b_ref,
               p_ref, cinv_ref):
    cnt = cnt_ref[...]
    rin = 1.0 / jnp.maximum(cnt, 1.0)
    nz = (cnt > 0.0).astype(_f32)
    M = (segp_ref[0] + segp_ref[1]) * rin
    x0 = jax.nn.relu(
        jnp.dot(M, Wi_ref[...], preferred_element_type=_f32)
        + nz * bit_ref[...] + bi_ref[...])
    p_ref[...] = jnp.dot(x0, W1b_ref[...], preferred_element_type=_f32)
    cinv_ref[...] = rin


def _prep(segp, cnt, Wi, bit, bi, W1b):
    return pl.pallas_call(
        _prep_body,
        out_shape=[
            jax.ShapeDtypeStruct((N, H), _f32),
            jax.ShapeDtypeStruct((N, 1), _f32),
        ],
    )(segp, cnt, Wi, bit, bi, W1b)


BM = 6400


def _msg_body(ea_ref, g_ref, W1_ref, b1_ref, W2_ref, b2_ref, out_ref):
    h = jax.nn.relu(_dotT(ea_ref[...], W1_ref[...]) + g_ref[...] + b1_ref[...])
    out_ref[...] = jax.nn.relu(
        jnp.dot(h, W2_ref[...], preferred_element_type=_f32) + b2_ref[...])


def _msg(eaT, g, W1ap, b1p, We2, be2, half):
    nb = E // BM // 2
    return pl.pallas_call(
        _msg_body,
        grid=(nb,),
        in_specs=[
            pl.BlockSpec((ED, BM), lambda i: (0, i + half * nb)),
            pl.BlockSpec((BM, H), lambda i: (i, 0)),
            pl.BlockSpec((ED, H), lambda i: (0, 0)),
            pl.BlockSpec((1, H), lambda i: (0, 0)),
            pl.BlockSpec((H, H), lambda i: (0, 0)),
            pl.BlockSpec((1, H), lambda i: (0, 0)),
        ],
        out_specs=pl.BlockSpec((BM, H), lambda i: (i, 0)),
        out_shape=jax.ShapeDtypeStruct((E // 2, H), _f32),
    )(eaT, g, W1ap, b1p, We2, be2)


def _update_body(aggp_ref, cinv_ref, Wu_ref, bu_ref, Wn_ref, bn_ref, out_ref):
    agg = (aggp_ref[0] + aggp_ref[1]) * cinv_ref[...]
    x = jax.nn.relu(
        jnp.dot(agg, Wu_ref[...], preferred_element_type=_f32) + bu_ref[...])
    out_ref[...] = jnp.dot(x, Wn_ref[...], preferred_element_type=_f32) + bn_ref[...]


def _update(aggp, cinv, Wu, bu, Wn, bn):
    return pl.pallas_call(
        _update_body,
        out_shape=jax.ShapeDtypeStruct((N, Wn.shape[1]), _f32),
    )(aggp, cinv, Wu, bu, Wn, bn)


def kernel(edge_index, edge_attr, bn_gamma, bn_beta, W_init, b_init,
           W_e1, b_e1, W_e2, b_e2, W_u, b_u, W_out, b_out):
    src1d = edge_index[0]
    dst3 = edge_index[1].reshape(ROWS, 1, 128)

    eaT = edge_attr.T
    ssum, ssq = _stats(eaT)
    mu = ssum.reshape(1, ED) / float(E)
    var = ssq.reshape(1, ED) / float(E) - mu * mu
    sv = bn_gamma[None, :] * lax.rsqrt(var + EPS)
    tv = bn_beta[None, :] - mu * sv
    W1a = W_e1[:ED]
    W1b = W_e1[ED:]
    W1ap = W1a * sv.reshape(ED, 1)
    b1p = tv @ W1a + b_e1[None, :]
    Wi = W_init * sv.reshape(ED, 1)
    bit = tv @ W_init

    zeros128 = jnp.zeros((N, H), _f32)
    zeros16 = jnp.zeros((N, ED), _f32)
    segp = _seg16(edge_attr, dst3, zeros16)
    cnts = _counts(edge_index[1])
    cnt = cnts.sum(axis=0).reshape(-1)[:N].astype(_f32).reshape(N, 1)

    p, cinv = _prep(segp, cnt, Wi, bit, b_init[None, :], W1b)

    be2 = b_e2[None, :]
    bu = b_u[None, :]
    for layer in range(NUM_LAYERS):
        g0 = _gather(p, src1d, 0)
        g1 = _gather(p, src1d, 1)
        m0 = _msg(eaT, g0, W1ap, b1p, W_e2, be2, 0)
        m1 = _msg(eaT, g1, W1ap, b1p, W_e2, be2, 1)
        aggp = _scatter(m0, m1, dst3, zeros128)
        if layer < NUM_LAYERS - 1:
            p = _update(aggp, cinv, W_u, bu, W1b, jnp.zeros((1, H), _f32))
        else:
            out = _update(aggp, cinv, W_u, bu, W_out, b_out[None, :])
    return out


# speedup vs baseline: 1.0323x; 1.0017x over previous
import functools

import jax
import jax.numpy as jnp
from jax import lax
from jax.experimental import pallas as pl
from jax.experimental.pallas import tpu as pltpu
from jax.experimental.pallas import tpu_sc as plsc

N = 10000
E = 320000
ED = 16
H = 128
OUT = 128
NUM_LAYERS = 3
EPS = 1e-5

NC = 2
NS = 16
NW = NC * NS
ROWS = E // 128
RPW = ROWS // NW
TAIL = ROWS - RPW * NW
STR = 624

_f32 = jnp.float32


def _mesh():
    return plsc.VectorSubcoreMesh(core_axis_name="c", subcore_axis_name="s")


def _striped_copy(s, src, dst):
    pltpu.sync_copy(src.at[pl.ds(s * STR, STR)], dst.at[pl.ds(s * STR, STR)])

    @pl.when(s == NS - 1)
    def _():
        rem = N - NS * STR
        pltpu.sync_copy(src.at[pl.ds(NS * STR, rem)], dst.at[pl.ds(NS * STR, rem)])


def _stats_body(ea_ref, sum_ref, sq_ref):
    i = pl.program_id(0)
    x = ea_ref[...]

    @pl.when(i == 0)
    def _():
        sum_ref[...] = jnp.zeros_like(sum_ref)
        sq_ref[...] = jnp.zeros_like(sq_ref)

    sum_ref[...] += jnp.sum(x, axis=1, keepdims=True)
    sq_ref[...] += jnp.sum(x * x, axis=1, keepdims=True)


def _stats(eaT):
    bs = 16000
    return pl.pallas_call(
        _stats_body,
        grid=(E // bs,),
        in_specs=[pl.BlockSpec((ED, bs), lambda i: (0, i))],
        out_specs=[pl.BlockSpec((ED, 1), lambda i: (0, 0))] * 2,
        out_shape=[jax.ShapeDtypeStruct((ED, 1), _f32)] * 2,
    )(eaT)


KG = 3
HROWS = ROWS // 2
GW = HROWS // NW
GTAIL = HROWS - GW * NW
GIPW = GW * 128


def _gather_body(p_hbm, src_hbm, g_hbm, gb0, gb1, ibufall, gsem, wsem, *, half):
    c = lax.axis_index("c")
    s = lax.axis_index("s")
    w = c * NS + s
    base = w * GW
    hoff = half * HROWS
    gbufs = (gb0, gb1)

    pltpu.sync_copy(src_hbm.at[pl.ds((hoff + base) * 128, GIPW)],
                    ibufall.at[pl.ds(0, GIPW)])

    @pl.when(w < GTAIL)
    def _():
        pltpu.sync_copy(src_hbm.at[pl.ds((hoff + NW * GW + w) * 128, 128)],
                        ibufall.at[pl.ds(GIPW, 128)])

    def g_descs(k, b):
        return [
            pltpu.make_async_copy(
                p_hbm.at[ibufall.at[pl.ds((k * KG + j) * 128, 128)]],
                gbufs[b].at[pl.ds(j * 128, 128)], gsem)
            for j in range(KG)
        ]

    def wb_desc(k, b):
        return pltpu.make_async_copy(
            gbufs[b], g_hbm.at[pl.ds((base + k * KG) * 128, KG * 128)], wsem)

    nch = GW // KG
    for k in range(nch):
        b = k % 2
        if k >= 2:
            wb_desc(k - 2, b).wait()
        descs = g_descs(k, b)
        for d in descs:
            d.start()
        for d in descs:
            d.wait()
        wb_desc(k, b).start()
    for k in (nch - 2, nch - 1):
        wb_desc(k, k % 2).wait()

    @pl.when(w < GTAIL)
    def _tail():
        r = NW * GW + w
        d = pltpu.make_async_copy(
            p_hbm.at[ibufall.at[pl.ds(GIPW, 128)]],
            gbufs[0].at[pl.ds(0, 128)], gsem)
        d.start()
        d.wait()
        pltpu.sync_copy(gbufs[0].at[pl.ds(0, 128)], g_hbm.at[pl.ds(r * 128, 128)])


def _gather(p, src1d, half):
    import functools as _ft
    fn = pl.kernel(
        _ft.partial(_gather_body, half=half),
        out_type=jax.ShapeDtypeStruct((HROWS * 128, H), _f32),
        mesh=_mesh(),
        scratch_types=[
            pltpu.VMEM((KG * 128, H), _f32),
            pltpu.VMEM((KG * 128, H), _f32),
            pltpu.VMEM((GIPW + 128,), jnp.int32),
            pltpu.SemaphoreType.DMA,
            pltpu.SemaphoreType.DMA,
        ],
    )
    return fn(p, src1d)


KC = 1
SW = HROWS // NS
STAILC = HROWS - SW * NS


def _scatter_body(msg0_hbm, msg1_hbm, dst3_hbm, zero128_hbm, agg_hbm,
                  mb0, mb1, ib0, ib1, acc, lsem, ssem):
    c = lax.axis_index("c")
    s = lax.axis_index("s")
    mbufs = (mb0, mb1)
    ibufs = (ib0, ib1)

    _striped_copy(s, zero128_hbm, acc)
    plsc.subcore_barrier()

    def emit(msg_hbm, row_base):
        base = s * SW

        def load_descs(k, b):
            return [
                pltpu.make_async_copy(
                    dst3_hbm.at[pl.ds(row_base + base + k, 1)], ibufs[b], lsem),
                pltpu.make_async_copy(
                    msg_hbm.at[pl.ds((base + k) * 128, 128)], mbufs[b], lsem),
            ]

        def add_desc(k, b):
            return pltpu.make_async_copy(mbufs[b], acc.at[ibufs[b].at[0, 0]], ssem)

        for b in range(2):
            for d in load_descs(b, b):
                d.start()

        @pl.loop(0, SW // 2)
        def _o(o):
            for b in range(2):
                k = o * 2 + b
                for d in load_descs(k, b):
                    d.wait()
                d = add_desc(k, b)
                d.start(add=True)
                d.wait()

                @pl.when(k + 2 < SW)
                def _():
                    for d2 in load_descs(k + 2, b):
                        d2.start()

        @pl.when(s < STAILC)
        def _tail():
            r = NS * SW + s
            pltpu.sync_copy(dst3_hbm.at[pl.ds(row_base + r, 1)], ib0)
            pltpu.sync_copy(msg_hbm.at[pl.ds(r * 128, 128)], mb0)
            pltpu.sync_copy(mb0, acc.at[ib0.at[0, 0]], add=True)

    @pl.when(c == 0)
    def _h0():
        emit(msg0_hbm, 0)

    @pl.when(c == 1)
    def _h1():
        emit(msg1_hbm, HROWS)

    plsc.subcore_barrier()
    _striped_copy(s, acc, agg_hbm.at[c])


def _scatter(msg0, msg1, dst3, zeros128):
    fn = pl.kernel(
        _scatter_body,
        out_type=jax.ShapeDtypeStruct((NC, N, H), _f32),
        mesh=_mesh(),
        scratch_types=[
            pltpu.VMEM((128, H), _f32),
            pltpu.VMEM((128, H), _f32),
            pltpu.VMEM((1, 1, 128), jnp.int32),
            pltpu.VMEM((1, 1, 128), jnp.int32),
            pltpu.VMEM_SHARED((N, H), _f32),
            pltpu.SemaphoreType.DMA,
            pltpu.SemaphoreType.DMA,
        ],
    )
    return fn(msg0, msg1, dst3, zeros128)


KS = 13


def _seg16_body(attr_hbm, dst3_hbm, zero16_hbm, outS_hbm, abuf, ibuf, accS):
    c = lax.axis_index("c")
    s = lax.axis_index("s")
    w = c * NS + s
    _striped_copy(s, zero16_hbm, accS)
    plsc.subcore_barrier()
    base = w * RPW

    @pl.loop(0, RPW // KS)
    def _chunk(k):
        r0 = base + k * KS
        pltpu.sync_copy(dst3_hbm.at[pl.ds(r0, KS)], ibuf)
        pltpu.sync_copy(attr_hbm.at[pl.ds(r0 * 128, KS * 128)], abuf)
        for j in range(KS):
            pltpu.sync_copy(abuf.at[pl.ds(j * 128, 128)], accS.at[ibuf.at[j, 0]], add=True)

    @pl.when(w < TAIL)
    def _tail():
        r = NW * RPW + w
        pltpu.sync_copy(dst3_hbm.at[pl.ds(r, 1)], ibuf.at[pl.ds(0, 1)])
        pltpu.sync_copy(attr_hbm.at[pl.ds(r * 128, 128)], abuf.at[pl.ds(0, 128)])
        pltpu.sync_copy(abuf.at[pl.ds(0, 128)], accS.at[ibuf.at[0, 0]], add=True)

    plsc.subcore_barrier()
    _striped_copy(s, accS, outS_hbm.at[c])


def _seg16(edge_attr, dst3, zeros16):
    fn = pl.kernel(
        _seg16_body,
        out_type=jax.ShapeDtypeStruct((NC, N, ED), _f32),
        mesh=_mesh(),
        scratch_types=[
            pltpu.VMEM((KS * 128, ED), _f32),
            pltpu.VMEM((KS, 1, 128), jnp.int32),
            pltpu.VMEM_SHARED((N, ED), _f32),
        ],
        compiler_params=pltpu.CompilerParams(use_tc_tiling_on_sc=False),
    )
    return fn(edge_attr, dst3, zeros16)


CR = 8
CW = 1280
HN = CR * CW // 2
IPW = RPW * 128


def _counts_body(dst_hbm, cnt_hbm, ibufall, cbuf, obuf):
    c = lax.axis_index("c")
    s = lax.axis_index("s")
    w = c * NS + s
    lane = lax.iota(jnp.int32, 16)
    ones = jnp.ones((16,), jnp.int32)
    zeros = jnp.zeros((16,), jnp.int32)

    pltpu.sync_copy(dst_hbm.at[pl.ds(w * IPW, IPW)], ibufall.at[pl.ds(0, IPW)])

    @pl.when(w < TAIL)
    def _():
        pltpu.sync_copy(dst_hbm.at[pl.ds(NW * IPW + w * 128, 128)],
                        ibufall.at[pl.ds(IPW, 128)])

    for half in range(2):
        lo = half * HN

        @pl.loop(0, HN, unroll=8)
        def _zero(i):
            cbuf[pl.ds(i * 16, 16)] = zeros

        def _count(i):
            idx = ibufall[pl.ds(i * 16, 16)]
            rel = idx - lo
            m = (rel >= 0) & (rel < HN)
            addr = rel * 16 + lane
            plsc.addupdate_scatter(cbuf, [addr], ones, mask=m)

        pl.loop(0, IPW // 16, unroll=4)(_count)

        @pl.when(w < TAIL)
        def _count_tail():
            pl.loop(IPW // 16, (IPW + 128) // 16)(_count)

        @pl.loop(0, HN // 16, unroll=2)
        def _reduce(gi):
            n0 = gi * 16
            base = n0 * 16 + lane * 16
            acc = plsc.load_gather(cbuf, [base])
            for l in range(1, 16):
                acc = acc + plsc.load_gather(cbuf, [base + l])
            flat = lo + n0
            obuf[flat // CW, pl.ds(flat % CW, 16)] = acc

    pltpu.sync_copy(obuf, cnt_hbm.at[w])


def _counts(dst1d):
    fn = pl.kernel(
        _counts_body,
        out_type=jax.ShapeDtypeStruct((NW, CR, CW), jnp.int32),
        mesh=_mesh(),
        scratch_types=[
            pltpu.VMEM((IPW + 128,), jnp.int32),
            pltpu.VMEM((HN * 16,), jnp.int32),
            pltpu.VMEM((CR, CW), jnp.int32),
        ],
        compiler_params=pltpu.CompilerParams(needs_layout_passes=False),
    )
    return fn(dst1d)


def _dotT(a, b):
    return lax.dot_general(a, b, ((((0,), (0,))), ((), ())),
                           preferred_element_type=_f32)


def _qinit_body(ea_ref, Wi_ref, q_ref):
    q_ref[...] = _dotT(ea_ref[...], Wi_ref[...])


def _qinit(eaT, Wi, half):
    nb = E // BM // 2
    return pl.pallas_call(
        _qinit_body,
        grid=(nb,),
        in_specs=[
            pl.BlockSpec((ED, BM), lambda i: (0, i + half * nb)),
            pl.BlockSpec((ED, H), lambda i: (0, 0)),
        ],
        out_specs=pl.BlockSpec((BM, H), lambda i: (i, 0)),
        out_shape=jax.ShapeDtypeStruct((E // 2, H), _f32),
    )(eaT, Wi)


def _prep_body(segp_ref, cnt_ref, Wi_ref, bit_ref, bi_ref, W1b_ref,
               p_ref, cinv_ref):
    cnt = cnt_ref[...]
    rin = 1.0 / jnp.maximum(cnt, 1.0)
    nz = (cnt > 0.0).astype(_f32)
    M = (segp_ref[0] + segp_ref[1]) * rin
    x0 = jax.nn.relu(
        jnp.dot(M, Wi_ref[...], preferred_element_type=_f32)
        + nz * bit_ref[...] + bi_ref[...])
    p_ref[...] = jnp.dot(x0, W1b_ref[...], preferred_element_type=_f32)
    cinv_ref[...] = rin


def _prep(segp, cnt, Wi, bit, bi, W1b):
    return pl.pallas_call(
        _prep_body,
        out_shape=[
            jax.ShapeDtypeStruct((N, H), _f32),
            jax.ShapeDtypeStruct((N, 1), _f32),
        ],
    )(segp, cnt, Wi, bit, bi, W1b)


BM = 6400


def _msg_body(ea_ref, g_ref, W1_ref, b1_ref, W2_ref, b2_ref, out_ref):
    h = jax.nn.relu(_dotT(ea_ref[...], W1_ref[...]) + g_ref[...] + b1_ref[...])
    out_ref[...] = jax.nn.relu(
        jnp.dot(h.astype(jnp.bfloat16), W2_ref[...],
                preferred_element_type=_f32) + b2_ref[...])


def _msg(eaT, g, W1ap, b1p, We2, be2, half):
    nb = E // BM // 2
    return pl.pallas_call(
        _msg_body,
        grid=(nb,),
        in_specs=[
            pl.BlockSpec((ED, BM), lambda i: (0, i + half * nb)),
            pl.BlockSpec((BM, H), lambda i: (i, 0)),
            pl.BlockSpec((ED, H), lambda i: (0, 0)),
            pl.BlockSpec((1, H), lambda i: (0, 0)),
            pl.BlockSpec((H, H), lambda i: (0, 0)),
            pl.BlockSpec((1, H), lambda i: (0, 0)),
        ],
        out_specs=pl.BlockSpec((BM, H), lambda i: (i, 0)),
        out_shape=jax.ShapeDtypeStruct((E // 2, H), _f32),
    )(eaT, g, W1ap, b1p, We2.astype(jnp.bfloat16), be2)


def _update_body(aggp_ref, cinv_ref, Wu_ref, bu_ref, Wn_ref, bn_ref, out_ref):
    agg = (aggp_ref[0] + aggp_ref[1]) * cinv_ref[...]
    x = jax.nn.relu(
        jnp.dot(agg, Wu_ref[...], preferred_element_type=_f32) + bu_ref[...])
    out_ref[...] = jnp.dot(x, Wn_ref[...], preferred_element_type=_f32) + bn_ref[...]


def _update(aggp, cinv, Wu, bu, Wn, bn):
    return pl.pallas_call(
        _update_body,
        out_shape=jax.ShapeDtypeStruct((N, Wn.shape[1]), _f32),
    )(aggp, cinv, Wu, bu, Wn, bn)


def kernel(edge_index, edge_attr, bn_gamma, bn_beta, W_init, b_init,
           W_e1, b_e1, W_e2, b_e2, W_u, b_u, W_out, b_out):
    src1d = edge_index[0]
    dst3 = edge_index[1].reshape(ROWS, 1, 128)

    eaT = edge_attr.T
    ssum, ssq = _stats(eaT)
    mu = ssum.reshape(1, ED) / float(E)
    var = ssq.reshape(1, ED) / float(E) - mu * mu
    sv = bn_gamma[None, :] * lax.rsqrt(var + EPS)
    tv = bn_beta[None, :] - mu * sv
    W1a = W_e1[:ED]
    W1b = W_e1[ED:]
    W1ap = W1a * sv.reshape(ED, 1)
    b1p = tv @ W1a + b_e1[None, :]
    Wi = W_init * sv.reshape(ED, 1)
    bit = tv @ W_init

    zeros128 = jnp.zeros((N, H), _f32)
    zeros16 = jnp.zeros((N, ED), _f32)
    segp = _seg16(edge_attr, dst3, zeros16)
    cnts = _counts(edge_index[1])
    cnt = cnts.sum(axis=0).reshape(-1)[:N].astype(_f32).reshape(N, 1)

    p, cinv = _prep(segp, cnt, Wi, bit, b_init[None, :], W1b)

    be2 = b_e2[None, :]
    bu = b_u[None, :]
    for layer in range(NUM_LAYERS):
        g0 = _gather(p, src1d, 0)
        g1 = _gather(p, src1d, 1)
        m0 = _msg(eaT, g0, W1ap, b1p, W_e2, be2, 0)
        m1 = _msg(eaT, g1, W1ap, b1p, W_e2, be2, 1)
        aggp = _scatter(m0, m1, dst3, zeros128)
        if layer < NUM_LAYERS - 1:
            p = _update(aggp, cinv, W_u, bu, W1b, jnp.zeros((1, H), _f32))
        else:
            out = _update(aggp, cinv, W_u, bu, W_out, b_out[None, :])
    return out


# speedup vs baseline: 1.0368x; 1.0044x over previous
import functools

import jax
import jax.numpy as jnp
from jax import lax
from jax.experimental import pallas as pl
from jax.experimental.pallas import tpu as pltpu
from jax.experimental.pallas import tpu_sc as plsc

N = 10000
E = 320000
ED = 16
H = 128
OUT = 128
NUM_LAYERS = 3
EPS = 1e-5

NC = 2
NS = 16
NW = NC * NS
ROWS = E // 128
RPW = ROWS // NW
TAIL = ROWS - RPW * NW
STR = 624

_f32 = jnp.float32


def _mesh():
    return plsc.VectorSubcoreMesh(core_axis_name="c", subcore_axis_name="s")


def _striped_copy(s, src, dst):
    pltpu.sync_copy(src.at[pl.ds(s * STR, STR)], dst.at[pl.ds(s * STR, STR)])

    @pl.when(s == NS - 1)
    def _():
        rem = N - NS * STR
        pltpu.sync_copy(src.at[pl.ds(NS * STR, rem)], dst.at[pl.ds(NS * STR, rem)])


def _stats_body(ea_ref, sum_ref, sq_ref):
    i = pl.program_id(0)
    x = ea_ref[...]

    @pl.when(i == 0)
    def _():
        sum_ref[...] = jnp.zeros_like(sum_ref)
        sq_ref[...] = jnp.zeros_like(sq_ref)

    sum_ref[...] += jnp.sum(x, axis=1, keepdims=True)
    sq_ref[...] += jnp.sum(x * x, axis=1, keepdims=True)


def _stats(eaT):
    bs = 16000
    return pl.pallas_call(
        _stats_body,
        grid=(E // bs,),
        in_specs=[pl.BlockSpec((ED, bs), lambda i: (0, i))],
        out_specs=[pl.BlockSpec((ED, 1), lambda i: (0, 0))] * 2,
        out_shape=[jax.ShapeDtypeStruct((ED, 1), _f32)] * 2,
    )(eaT)


KG = 3
HROWS = ROWS // 2
GW = HROWS // NW
GTAIL = HROWS - GW * NW
GIPW = GW * 128


def _gather_body(p_hbm, src_hbm, g_hbm, gb0, gb1, ibufall, gsem, wsem, *, half):
    c = lax.axis_index("c")
    s = lax.axis_index("s")
    w = c * NS + s
    base = w * GW
    hoff = half * HROWS
    gbufs = (gb0, gb1)

    pltpu.sync_copy(src_hbm.at[pl.ds((hoff + base) * 128, GIPW)],
                    ibufall.at[pl.ds(0, GIPW)])

    @pl.when(w < GTAIL)
    def _():
        pltpu.sync_copy(src_hbm.at[pl.ds((hoff + NW * GW + w) * 128, 128)],
                        ibufall.at[pl.ds(GIPW, 128)])

    def g_descs(k, b):
        return [
            pltpu.make_async_copy(
                p_hbm.at[ibufall.at[pl.ds((k * KG + j) * 128, 128)]],
                gbufs[b].at[pl.ds(j * 128, 128)], gsem)
            for j in range(KG)
        ]

    def wb_desc(k, b):
        return pltpu.make_async_copy(
            gbufs[b], g_hbm.at[pl.ds((base + k * KG) * 128, KG * 128)], wsem)

    nch = GW // KG
    for k in range(nch):
        b = k % 2
        if k >= 2:
            wb_desc(k - 2, b).wait()
        descs = g_descs(k, b)
        for d in descs:
            d.start()
        for d in descs:
            d.wait()
        wb_desc(k, b).start()
    for k in (nch - 2, nch - 1):
        wb_desc(k, k % 2).wait()

    @pl.when(w < GTAIL)
    def _tail():
        r = NW * GW + w
        d = pltpu.make_async_copy(
            p_hbm.at[ibufall.at[pl.ds(GIPW, 128)]],
            gbufs[0].at[pl.ds(0, 128)], gsem)
        d.start()
        d.wait()
        pltpu.sync_copy(gbufs[0].at[pl.ds(0, 128)], g_hbm.at[pl.ds(r * 128, 128)])


def _gather(p, src1d, half):
    import functools as _ft
    fn = pl.kernel(
        _ft.partial(_gather_body, half=half),
        out_type=jax.ShapeDtypeStruct((HROWS * 128, H), _f32),
        mesh=_mesh(),
        scratch_types=[
            pltpu.VMEM((KG * 128, H), _f32),
            pltpu.VMEM((KG * 128, H), _f32),
            pltpu.VMEM((GIPW + 128,), jnp.int32),
            pltpu.SemaphoreType.DMA,
            pltpu.SemaphoreType.DMA,
        ],
    )
    return fn(p, src1d)


KC = 1
SW = HROWS // NS
STAILC = HROWS - SW * NS


def _scatter_body(msg0_hbm, msg1_hbm, dst3_hbm, zero128_hbm, agg_hbm,
                  mb0, mb1, ib0, ib1, acc, lsem, ssem):
    c = lax.axis_index("c")
    s = lax.axis_index("s")
    mbufs = (mb0, mb1)
    ibufs = (ib0, ib1)

    def emit(msg_hbm, row_base):
        base = s * SW

        def load_descs(k, b):
            return [
                pltpu.make_async_copy(
                    dst3_hbm.at[pl.ds(row_base + base + k, 1)], ibufs[b], lsem),
                pltpu.make_async_copy(
                    msg_hbm.at[pl.ds((base + k) * 128, 128)], mbufs[b], lsem),
            ]

        def add_desc(k, b):
            return pltpu.make_async_copy(mbufs[b], acc.at[ibufs[b].at[0, 0]], ssem)

        for b in range(2):
            for d in load_descs(b, b):
                d.start()

        _striped_copy(s, zero128_hbm, acc)
        plsc.subcore_barrier()

        @pl.loop(0, SW // 2)
        def _o(o):
            for b in range(2):
                k = o * 2 + b
                for d in load_descs(k, b):
                    d.wait()
                d = add_desc(k, b)
                d.start(add=True)
                d.wait()

                @pl.when(k + 2 < SW)
                def _():
                    for d2 in load_descs(k + 2, b):
                        d2.start()

        @pl.when(s < STAILC)
        def _tail():
            r = NS * SW + s
            pltpu.sync_copy(dst3_hbm.at[pl.ds(row_base + r, 1)], ib0)
            pltpu.sync_copy(msg_hbm.at[pl.ds(r * 128, 128)], mb0)
            pltpu.sync_copy(mb0, acc.at[ib0.at[0, 0]], add=True)

    @pl.when(c == 0)
    def _h0():
        emit(msg0_hbm, 0)

    @pl.when(c == 1)
    def _h1():
        emit(msg1_hbm, HROWS)

    plsc.subcore_barrier()
    _striped_copy(s, acc, agg_hbm.at[c])


def _scatter(msg0, msg1, dst3, zeros128):
    fn = pl.kernel(
        _scatter_body,
        out_type=jax.ShapeDtypeStruct((NC, N, H), _f32),
        mesh=_mesh(),
        scratch_types=[
            pltpu.VMEM((128, H), _f32),
            pltpu.VMEM((128, H), _f32),
            pltpu.VMEM((1, 1, 128), jnp.int32),
            pltpu.VMEM((1, 1, 128), jnp.int32),
            pltpu.VMEM_SHARED((N, H), _f32),
            pltpu.SemaphoreType.DMA,
            pltpu.SemaphoreType.DMA,
        ],
    )
    return fn(msg0, msg1, dst3, zeros128)


KS = 13


def _seg16_body(attr_hbm, dst3_hbm, zero16_hbm, outS_hbm, abuf, ibuf, accS):
    c = lax.axis_index("c")
    s = lax.axis_index("s")
    w = c * NS + s
    _striped_copy(s, zero16_hbm, accS)
    plsc.subcore_barrier()
    base = w * RPW

    @pl.loop(0, RPW // KS)
    def _chunk(k):
        r0 = base + k * KS
        pltpu.sync_copy(dst3_hbm.at[pl.ds(r0, KS)], ibuf)
        pltpu.sync_copy(attr_hbm.at[pl.ds(r0 * 128, KS * 128)], abuf)
        for j in range(KS):
            pltpu.sync_copy(abuf.at[pl.ds(j * 128, 128)], accS.at[ibuf.at[j, 0]], add=True)

    @pl.when(w < TAIL)
    def _tail():
        r = NW * RPW + w
        pltpu.sync_copy(dst3_hbm.at[pl.ds(r, 1)], ibuf.at[pl.ds(0, 1)])
        pltpu.sync_copy(attr_hbm.at[pl.ds(r * 128, 128)], abuf.at[pl.ds(0, 128)])
        pltpu.sync_copy(abuf.at[pl.ds(0, 128)], accS.at[ibuf.at[0, 0]], add=True)

    plsc.subcore_barrier()
    _striped_copy(s, accS, outS_hbm.at[c])


def _seg16(edge_attr, dst3, zeros16):
    fn = pl.kernel(
        _seg16_body,
        out_type=jax.ShapeDtypeStruct((NC, N, ED), _f32),
        mesh=_mesh(),
        scratch_types=[
            pltpu.VMEM((KS * 128, ED), _f32),
            pltpu.VMEM((KS, 1, 128), jnp.int32),
            pltpu.VMEM_SHARED((N, ED), _f32),
        ],
        compiler_params=pltpu.CompilerParams(use_tc_tiling_on_sc=False),
    )
    return fn(edge_attr, dst3, zeros16)


CR = 8
CW = 1280
HN = CR * CW // 2
IPW = RPW * 128


def _counts_body(dst_hbm, cnt_hbm, ibufall, cbuf, obuf):
    c = lax.axis_index("c")
    s = lax.axis_index("s")
    w = c * NS + s
    lane = lax.iota(jnp.int32, 16)
    ones = jnp.ones((16,), jnp.int32)
    zeros = jnp.zeros((16,), jnp.int32)

    pltpu.sync_copy(dst_hbm.at[pl.ds(w * IPW, IPW)], ibufall.at[pl.ds(0, IPW)])

    @pl.when(w < TAIL)
    def _():
        pltpu.sync_copy(dst_hbm.at[pl.ds(NW * IPW + w * 128, 128)],
                        ibufall.at[pl.ds(IPW, 128)])

    for half in range(2):
        lo = half * HN

        @pl.loop(0, HN, unroll=8)
        def _zero(i):
            cbuf[pl.ds(i * 16, 16)] = zeros

        def _count(i):
            idx = ibufall[pl.ds(i * 16, 16)]
            rel = idx - lo
            m = (rel >= 0) & (rel < HN)
            addr = rel * 16 + lane
            plsc.addupdate_scatter(cbuf, [addr], ones, mask=m)

        pl.loop(0, IPW // 16, unroll=4)(_count)

        @pl.when(w < TAIL)
        def _count_tail():
            pl.loop(IPW // 16, (IPW + 128) // 16)(_count)

        @pl.loop(0, HN // 16, unroll=2)
        def _reduce(gi):
            n0 = gi * 16
            base = n0 * 16 + lane * 16
            acc = plsc.load_gather(cbuf, [base])
            for l in range(1, 16):
                acc = acc + plsc.load_gather(cbuf, [base + l])
            flat = lo + n0
            obuf[flat // CW, pl.ds(flat % CW, 16)] = acc

    pltpu.sync_copy(obuf, cnt_hbm.at[w])


def _counts(dst1d):
    fn = pl.kernel(
        _counts_body,
        out_type=jax.ShapeDtypeStruct((NW, CR, CW), jnp.int32),
        mesh=_mesh(),
        scratch_types=[
            pltpu.VMEM((IPW + 128,), jnp.int32),
            pltpu.VMEM((HN * 16,), jnp.int32),
            pltpu.VMEM((CR, CW), jnp.int32),
        ],
        compiler_params=pltpu.CompilerParams(needs_layout_passes=False),
    )
    return fn(dst1d)


def _dotT(a, b):
    return lax.dot_general(a, b, ((((0,), (0,))), ((), ())),
                           preferred_element_type=_f32)


def _prep_body(segp_ref, cnt_ref, Wi_ref, bit_ref, bi_ref, W1b_ref,
               p_ref, cinv_ref):
    cnt = cnt_ref[...]
    rin = 1.0 / jnp.maximum(cnt, 1.0)
    nz = (cnt > 0.0).astype(_f32)
    M = (segp_ref[0] + segp_ref[1]) * rin
    x0 = jax.nn.relu(
        jnp.dot(M, Wi_ref[...], preferred_element_type=_f32)
        + nz * bit_ref[...] + bi_ref[...])
    p_ref[...] = jnp.dot(x0, W1b_ref[...], preferred_element_type=_f32)
    cinv_ref[...] = rin


def _prep(segp, cnt, Wi, bit, bi, W1b):
    return pl.pallas_call(
        _prep_body,
        out_shape=[
            jax.ShapeDtypeStruct((N, H), _f32),
            jax.ShapeDtypeStruct((N, 1), _f32),
        ],
    )(segp, cnt, Wi, bit, bi, W1b)


BM = 6400


def _msg_body(ea_ref, g_ref, W1_ref, b1_ref, W2_ref, b2_ref, out_ref):
    h = jax.nn.relu(_dotT(ea_ref[...], W1_ref[...]) + g_ref[...] + b1_ref[...])
    out_ref[...] = jax.nn.relu(
        jnp.dot(h.astype(jnp.bfloat16), W2_ref[...],
                preferred_element_type=_f32) + b2_ref[...])


def _msg(eaT, g, W1ap, b1p, We2, be2, half):
    nb = E // BM // 2
    return pl.pallas_call(
        _msg_body,
        grid=(nb,),
        in_specs=[
            pl.BlockSpec((ED, BM), lambda i: (0, i + half * nb)),
            pl.BlockSpec((BM, H), lambda i: (i, 0)),
            pl.BlockSpec((ED, H), lambda i: (0, 0)),
            pl.BlockSpec((1, H), lambda i: (0, 0)),
            pl.BlockSpec((H, H), lambda i: (0, 0)),
            pl.BlockSpec((1, H), lambda i: (0, 0)),
        ],
        out_specs=pl.BlockSpec((BM, H), lambda i: (i, 0)),
        out_shape=jax.ShapeDtypeStruct((E // 2, H), _f32),
    )(eaT, g, W1ap, b1p, We2.astype(jnp.bfloat16), be2)


def _update_body(aggp_ref, cinv_ref, Wu_ref, bu_ref, Wn_ref, bn_ref, out_ref):
    agg = (aggp_ref[0] + aggp_ref[1]) * cinv_ref[...]
    x = jax.nn.relu(
        jnp.dot(agg, Wu_ref[...], preferred_element_type=_f32) + bu_ref[...])
    out_ref[...] = jnp.dot(x, Wn_ref[...], preferred_element_type=_f32) + bn_ref[...]


def _update(aggp, cinv, Wu, bu, Wn, bn):
    return pl.pallas_call(
        _update_body,
        out_shape=jax.ShapeDtypeStruct((N, Wn.shape[1]), _f32),
    )(aggp, cinv, Wu, bu, Wn, bn)


def kernel(edge_index, edge_attr, bn_gamma, bn_beta, W_init, b_init,
           W_e1, b_e1, W_e2, b_e2, W_u, b_u, W_out, b_out):
    src1d = edge_index[0]
    dst3 = edge_index[1].reshape(ROWS, 1, 128)

    eaT = edge_attr.T
    ssum, ssq = _stats(eaT)
    mu = ssum.reshape(1, ED) / float(E)
    var = ssq.reshape(1, ED) / float(E) - mu * mu
    sv = bn_gamma[None, :] * lax.rsqrt(var + EPS)
    tv = bn_beta[None, :] - mu * sv
    W1a = W_e1[:ED]
    W1b = W_e1[ED:]
    W1ap = W1a * sv.reshape(ED, 1)
    b1p = tv @ W1a + b_e1[None, :]
    Wi = W_init * sv.reshape(ED, 1)
    bit = tv @ W_init

    zeros128 = jnp.zeros((N, H), _f32)
    zeros16 = jnp.zeros((N, ED), _f32)
    segp = _seg16(edge_attr, dst3, zeros16)
    cnts = _counts(edge_index[1])
    cnt = cnts.sum(axis=0).reshape(-1)[:N].astype(_f32).reshape(N, 1)

    p, cinv = _prep(segp, cnt, Wi, bit, b_init[None, :], W1b)

    be2 = b_e2[None, :]
    bu = b_u[None, :]
    for layer in range(NUM_LAYERS):
        g0 = _gather(p, src1d, 0)
        g1 = _gather(p, src1d, 1)
        m0 = _msg(eaT, g0, W1ap, b1p, W_e2, be2, 0)
        m1 = _msg(eaT, g1, W1ap, b1p, W_e2, be2, 1)
        aggp = _scatter(m0, m1, dst3, zeros128)
        if layer < NUM_LAYERS - 1:
            p = _update(aggp, cinv, W_u, bu, W1b, jnp.zeros((1, H), _f32))
        else:
            out = _update(aggp, cinv, W_u, bu, W_out, b_out[None, :])
    return out
